# baseline (speedup 1.0000x reference)
import jax
import jax.numpy as jnp
from jax import lax
from jax.experimental import pallas as pl
from jax.experimental.pallas import tpu as pltpu
from jax.experimental.pallas import tpu_sc as plsc

_N = 10000
_E = 80000
_R = 4
_D = 128
_NB = 2

_NC = 2
_NS = 16
_LANES = 16

_CHUNK = 128
_EPAD = 81920
_EPT = _R * _EPAD // _NS
_NCHK = _EPT // _CHUNK
_NPAD = 10240
_ROWS_PT = _NPAD // _NS
_QD = 4


def _sc_body(inp_hbm, sdx_hbm, val_hbm, coeff_hbm, acc_hbm,
             acc_sp, sdx_b, val_b, rows0, rows1, coeff_v,
             g0, g1, s0, s1, i0, i1):
    c = lax.axis_index("c")
    s = lax.axis_index("s")

    pltpu.sync_copy(coeff_hbm, coeff_v)

    def _zrow(i, carry):
        for j in range(_D // _LANES):
            rows0[i, pl.ds(j * _LANES, _LANES)] = jnp.zeros((_LANES,), jnp.float32)
        return carry
    lax.fori_loop(0, _CHUNK, _zrow, 0)
    for t in range(_ROWS_PT // _CHUNK):
        pltpu.sync_copy(rows0, acc_sp.at[pl.ds(s * _ROWS_PT + t * _CHUNK, _CHUNK)])

    rel = s // (_NS // _R)
    cv = coeff_v[...]
    want = jnp.full((_LANES,), rel * _NB + c, jnp.int32)
    cvec = cv.at[want].get(mode="promise_in_bounds")

    plsc.subcore_barrier()

    def _ld_idx(k, sem):
        q = lax.rem(k, _QD)
        pltpu.async_copy(sdx_hbm.at[s, k], sdx_b.at[q], sem)
        pltpu.async_copy(val_hbm.at[s, k], val_b.at[q], sem)

    def _wait_idx(sem):
        pltpu.make_async_copy(sdx_hbm.at[s, 0], sdx_b.at[0], sem).wait()
        pltpu.make_async_copy(val_hbm.at[s, 0], val_b.at[0], sem).wait()

    def _gather(k, buf, sem):
        q = lax.rem(k, _QD)
        pltpu.async_copy(inp_hbm.at[sdx_b.at[q, 0]], buf, sem)

    def _wait_gather(buf, sem):
        pltpu.make_async_copy(inp_hbm.at[sdx_b.at[0, 0]], buf, sem).wait()

    def _scale(buf, k):
        q = lax.rem(k, _QD)

        def _g(g, carry):
            w16 = val_b[q, pl.ds(g * _LANES, _LANES)] * cvec
            for l in range(_LANES):
                w = w16[l]
                e = g * _LANES + l
                for j in range(_D // _LANES):
                    sl = pl.ds(j * _LANES, _LANES)
                    buf[e, sl] = buf[e, sl] * w
            return carry
        lax.fori_loop(0, _CHUNK // _LANES, _g, 0)

    def _scatter(k, buf, sem):
        q = lax.rem(k, _QD)
        return pltpu.async_copy(buf, acc_sp.at[sdx_b.at[q, 1]], sem, add=True)

    pltpu.sync_copy(sdx_hbm.at[s, 0], sdx_b.at[0])
    pltpu.sync_copy(val_hbm.at[s, 0], val_b.at[0])
    pltpu.sync_copy(sdx_hbm.at[s, 1], sdx_b.at[1])
    pltpu.sync_copy(val_hbm.at[s, 1], val_b.at[1])
    _gather(0, rows0, g0)
    _gather(1, rows1, g1)
    _ld_idx(jnp.int32(2), i0)
    _ld_idx(jnp.int32(3), i1)

    def _pair(m, carry):
        a = 2 * m
        b = a + 1
        _wait_gather(rows0, g0)
        _scale(rows0, a)
        sc0 = _scatter(a, rows0, s0)
        _wait_gather(rows1, g1)
        _scale(rows1, b)
        sc1 = _scatter(b, rows1, s1)
        sc0.wait()
        _wait_idx(i0)
        _gather(jnp.minimum(a + 2, _NCHK - 1), rows0, g0)
        _ld_idx(jnp.minimum(a + 4, _NCHK - 1), i0)
        sc1.wait()
        _wait_idx(i1)
        _gather(jnp.minimum(b + 2, _NCHK - 1), rows1, g1)
        _ld_idx(jnp.minimum(b + 4, _NCHK - 1), i1)
        return carry
    lax.fori_loop(0, _NCHK // 2, _pair, 0)

    _wait_gather(rows0, g0)
    _wait_gather(rows1, g1)
    _wait_idx(i0)
    _wait_idx(i1)

    plsc.subcore_barrier()
    rsl = pl.ds(s * _ROWS_PT, _ROWS_PT)
    pltpu.sync_copy(acc_sp.at[rsl], acc_hbm.at[c, rsl])


@jax.jit
def _sc_call(inp, sdx, val, coeff_flat):
    mesh = plsc.VectorSubcoreMesh(core_axis_name="c", subcore_axis_name="s",
                                  num_cores=_NC, num_subcores=_NS)
    return pl.kernel(
        _sc_body,
        out_type=jax.ShapeDtypeStruct((_NB, _NPAD, _D), jnp.float32),
        mesh=mesh,
        scratch_types=[
            pltpu.VMEM_SHARED((_NPAD, _D), jnp.float32),
            pltpu.VMEM((_QD, 2, _CHUNK), jnp.int32),
            pltpu.VMEM((_QD, _CHUNK), jnp.float32),
            pltpu.VMEM((_CHUNK, _D), jnp.float32),
            pltpu.VMEM((_CHUNK, _D), jnp.float32),
            pltpu.VMEM((_LANES,), jnp.float32),
            pltpu.SemaphoreType.DMA,
            pltpu.SemaphoreType.DMA,
            pltpu.SemaphoreType.DMA,
            pltpu.SemaphoreType.DMA,
            pltpu.SemaphoreType.DMA,
            pltpu.SemaphoreType.DMA,
        ],
    )(inp, sdx, val, coeff_flat)


_BLK = 2000


def _tc_body(acc_ref, bw_ref, bias_ref, out_ref):
    a0 = acc_ref[0]
    a1 = acc_ref[1]
    out = jnp.dot(a0, bw_ref[0], preferred_element_type=jnp.float32)
    out = out + jnp.dot(a1, bw_ref[1], preferred_element_type=jnp.float32)
    out_ref[...] = out + jnp.sum(bias_ref[...], axis=0)[None, :]


@jax.jit
def _tc_call(acc, basis_weights, bias):
    return pl.pallas_call(
        _tc_body,
        out_shape=jax.ShapeDtypeStruct((_N, _D), jnp.float32),
        grid=(_N // _BLK,),
        in_specs=[
            pl.BlockSpec((_NB, _BLK, _D), lambda i: (0, i, 0)),
            pl.BlockSpec((_NB, _D, _D), lambda i: (0, 0, 0)),
            pl.BlockSpec((_R, _D), lambda i: (0, 0)),
        ],
        out_specs=pl.BlockSpec((_BLK, _D), lambda i: (i, 0)),
    )(acc, basis_weights, bias)


def _edges3(x):
    xp = jnp.pad(x, ((0, 0), (0, _EPAD - _E)))
    return xp.reshape(_NS, _NCHK, _CHUNK)


def kernel(inp, edge_index, edge_val, basis_weights, basis_coeff, bias):
    dst = _edges3(edge_index[:, 0, :])
    src = _edges3(edge_index[:, 1, :])
    val = _edges3(edge_val)
    sdx = jnp.stack([src, dst], axis=2)
    coeff_flat = jnp.zeros((_LANES,), jnp.float32).at[: _R * _NB].set(
        basis_coeff.reshape(-1))
    acc = _sc_call(inp, sdx, val, coeff_flat)
    return _tc_call(acc, basis_weights, bias)
